# baseline (speedup 1.0000x reference)
import jax
import jax.numpy as jnp
from jax import lax
from jax.experimental import pallas as pl
from jax.experimental.pallas import tpu as pltpu
from jax.experimental.pallas import tpu_sc as plsc

NN = 10000
EE = 320000
NC, NS, LANES = 2, 16, 16
NW = NC * NS
ET = EE // NW
EB = 80
NB = ET // EB
CH = 25
RD = 624
TAIL = NN - NS * RD

f32 = jnp.float32
_PREC = lax.Precision.HIGHEST


def _sc_edge_pass(zs, zd, src3, dst3, zinit, rw, aoff, nv, vph):
    mesh = plsc.VectorSubcoreMesh(
        core_axis_name="c", subcore_axis_name="s",
        num_cores=NC, num_subcores=NS)

    def body(zs_hbm, zd_hbm, src_hbm, dst_hbm, zi_hbm, acc_hbm,
             acc_s, sidxc, didx, rows0, rows1, zdr0, zdr1,
             sg0, sg1, ss0, ss1):
        cid = lax.axis_index("c")
        sid = lax.axis_index("s")
        w = cid * NS + sid
        rows_ = (rows0, rows1)
        zdr_ = (zdr0, zdr1)
        sg_ = (sg0, sg1)
        ss_ = (ss0, ss1)

        ib = pltpu.async_copy(dst_hbm.at[w], didx, sg1)

        base_r = sid * RD
        pltpu.sync_copy(zi_hbm.at[pl.ds(0, RD), pl.ds(0, rw)],
                        acc_s.at[pl.ds(base_r, RD)])

        @pl.when(sid == 0)
        def _zero_tail():
            pltpu.sync_copy(zi_hbm.at[pl.ds(0, TAIL), pl.ds(0, rw)],
                            acc_s.at[pl.ds(NS * RD, TAIL)])
        pltpu.sync_copy(src_hbm.at[w, pl.ds(0, CH)], sidxc)
        ib.wait()
        plsc.subcore_barrier()

        heads = sorted(set(j // vph for j in range(nv)))
        hsplat = {h: jnp.full((LANES,), h, jnp.int32) for h in heads}

        def issue_gather(k, p):
            pltpu.async_copy(zs_hbm.at[sidxc.at[k % CH]], rows_[p], sg_[p])
            pltpu.async_copy(zd_hbm.at[didx.at[k]], zdr_[p], sg_[p])

        def wait_gather(p):
            pltpu.make_async_copy(zs_hbm.at[sidxc.at[0]], rows_[p],
                                  sg_[p]).wait()
            pltpu.make_async_copy(zd_hbm.at[didx.at[0]], zdr_[p],
                                  sg_[p]).wait()

        def issue_scatter(k, p):
            pltpu.async_copy(rows_[p], acc_s.at[didx.at[k]], ss_[p],
                             add=True)

        def wait_scatter(p):
            pltpu.make_async_copy(rows_[p], acc_s.at[didx.at[0]],
                                  ss_[p]).wait()

        def compute_half(p, h):
            rp = rows_[p]
            zp = zdr_[p]

            def edge1(b):
                av = rp[b, pl.ds(aoff, LANES)] + zp[b, :]
                av = jnp.where(av > 0, av, av * 0.2)
                sv = jnp.exp(av)
                rp[b, pl.ds(aoff, LANES)] = sv
                ms = {hh: sv.at[hsplat[hh]].get(mode="promise_in_bounds")
                      for hh in heads}
                for j in range(nv):
                    rp[b, pl.ds(j * LANES, LANES)] = (
                        ms[j // vph] * rp[b, pl.ds(j * LANES, LANES)])

            @plsc.parallel_loop(h * (EB // 2), (h + 1) * (EB // 2), unroll=4)
            def _edges(b):
                edge1(b)

        def step(k, p, issue_next, guard_first=False):
            wait_gather(p)
            compute_half(p, 0)
            if guard_first:
                @pl.when(k >= 1)
                def _w():
                    wait_scatter(1 - p)
            else:
                wait_scatter(1 - p)
            if issue_next:
                @pl.when((k + 1) % CH == 0)
                def _refill():
                    pltpu.sync_copy(src_hbm.at[w, pl.ds(k + 1, CH)], sidxc)
                issue_gather(k + 1, 1 - p)
            compute_half(p, 1)
            issue_scatter(k, p)

        issue_gather(0, 0)

        def superstep(s, carry):
            step(2 * s, 0, True, guard_first=True)
            step(2 * s + 1, 1, True)
            return carry
        lax.fori_loop(0, (NB - 1) // 2, superstep, 0)
        step(NB - 1, 0, False)
        wait_scatter(0)

        plsc.subcore_barrier()
        pltpu.sync_copy(acc_s.at[pl.ds(base_r, RD)],
                        acc_hbm.at[cid, pl.ds(base_r, RD)])

        @pl.when(sid == 0)
        def _drain_tail():
            pltpu.sync_copy(acc_s.at[pl.ds(NS * RD, TAIL)],
                            acc_hbm.at[cid, pl.ds(NS * RD, TAIL)])

    kfn = pl.kernel(
        body,
        out_type=jax.ShapeDtypeStruct((NC, NN, rw), f32),
        mesh=mesh,
        compiler_params=pltpu.CompilerParams(use_tc_tiling_on_sc=False),
        scratch_types=[
            pltpu.VMEM_SHARED((NN, rw), f32),
            pltpu.VMEM((CH, EB), jnp.int32),
            pltpu.VMEM((NB, EB), jnp.int32),
            pltpu.VMEM((EB, rw), f32),
            pltpu.VMEM((EB, rw), f32),
            pltpu.VMEM((EB, 16), f32),
            pltpu.VMEM((EB, 16), f32),
            pltpu.SemaphoreType.DMA,
            pltpu.SemaphoreType.DMA,
            pltpu.SemaphoreType.DMA,
            pltpu.SemaphoreType.DMA,
        ],
    )
    return kfn(zs, zd, src3, dst3, zinit)


def _tc_proj(x, wcat, wdp):
    k = wcat.shape[1]
    r = NN // 2

    def bdy(x_ref, wc_ref, wd_ref, z_ref, zd_ref):
        xb = x_ref[...]
        z_ref[...] = lax.dot_general(xb, wc_ref[...],
                                     (((1,), (0,)), ((), ())), precision=_PREC)
        zd_ref[...] = lax.dot_general(xb, wd_ref[...],
                                      (((1,), (0,)), ((), ())), precision=_PREC)

    return pl.pallas_call(
        bdy,
        grid=(NN // r,),
        in_specs=[pl.BlockSpec((r, x.shape[1]), lambda i: (i, 0)),
                  pl.BlockSpec(wcat.shape, lambda i: (0, 0)),
                  pl.BlockSpec(wdp.shape, lambda i: (0, 0))],
        out_specs=[pl.BlockSpec((r, k), lambda i: (i, 0)),
                   pl.BlockSpec((r, 16), lambda i: (i, 0))],
        out_shape=[jax.ShapeDtypeStruct((NN, k), f32),
                   jax.ShapeDtypeStruct((NN, 16), f32)],
    )(x, wcat, wdp)


def _tc_comb(acc, bias, dmat, wcat, wdp, hw, aoff):
    rwi = acc.shape[2]
    k = wcat.shape[1]
    r = NN // 2

    def bdy(a_ref, b_ref, dm_ref, wc_ref, wd_ref, z_ref, zd_ref):
        a = a_ref[0] + a_ref[1]
        h = a[:, :hw]
        d = a[:, aoff:aoff + 16]
        rec = 1.0 / (d + 1e-16)
        rb = lax.dot_general(rec, dm_ref[...],
                             (((1,), (0,)), ((), ())), precision=_PREC)
        xb = jnp.maximum(h * rb + b_ref[...], 0.0)
        z_ref[...] = lax.dot_general(xb, wc_ref[...],
                                     (((1,), (0,)), ((), ())), precision=_PREC)
        zd_ref[...] = lax.dot_general(xb, wd_ref[...],
                                      (((1,), (0,)), ((), ())), precision=_PREC)

    return pl.pallas_call(
        bdy,
        grid=(NN // r,),
        in_specs=[pl.BlockSpec((NC, r, rwi), lambda i: (0, i, 0)),
                  pl.BlockSpec((1, hw), lambda i: (0, 0)),
                  pl.BlockSpec((16, hw), lambda i: (0, 0)),
                  pl.BlockSpec(wcat.shape, lambda i: (0, 0)),
                  pl.BlockSpec(wdp.shape, lambda i: (0, 0))],
        out_specs=[pl.BlockSpec((r, k), lambda i: (i, 0)),
                   pl.BlockSpec((r, 16), lambda i: (i, 0))],
        out_shape=[jax.ShapeDtypeStruct((NN, k), f32),
                   jax.ShapeDtypeStruct((NN, 16), f32)],
    )(acc, bias, dmat, wcat, wdp)


def _tc_final(acc, bias, dmat, hw, aoff):
    rwi = acc.shape[2]
    r = NN // 2

    def bdy(a_ref, b_ref, dm_ref, o_ref):
        a = a_ref[0] + a_ref[1]
        h = a[:, :hw]
        d = a[:, aoff:aoff + 16]
        rec = 1.0 / (d + 1e-16)
        rb = lax.dot_general(rec, dm_ref[...],
                             (((1,), (0,)), ((), ())), precision=_PREC)
        o = h * rb + b_ref[...]
        m = jnp.max(o, axis=1, keepdims=True)
        e = o - m
        s = jnp.sum(jnp.exp(e), axis=1, keepdims=True)
        o_ref[...] = e - jnp.log(s)

    return pl.pallas_call(
        bdy,
        grid=(NN // r,),
        in_specs=[pl.BlockSpec((NC, r, rwi), lambda i: (0, i, 0)),
                  pl.BlockSpec((1, hw), lambda i: (0, 0)),
                  pl.BlockSpec((16, hw), lambda i: (0, 0))],
        out_specs=pl.BlockSpec((r, hw), lambda i: (i, 0)),
        out_shape=jax.ShapeDtypeStruct((NN, hw), f32),
    )(acc, bias, dmat)


def kernel(x, edge_index, W1, as1, ad1, b1, W2, as2, ad2, b2, W3, as3, ad3, b3):
    src3 = edge_index[0].reshape(NW, NB, EB)
    dst3 = edge_index[1].reshape(NW, NB, EB)

    gh = (jnp.arange(128)[:, None] // 16 == jnp.arange(8)[None, :]).astype(f32)

    def fold(W, a_s, a_d):
        asf = a_s.reshape(-1)
        adf = a_d.reshape(-1)
        ws = (W * asf[None, :]) @ gh
        wd = (W * adf[None, :]) @ gh
        return ws, wd

    z8 = jnp.zeros((128, 8), f32)
    ws1, wd1 = fold(W1, as1, ad1)
    wcat1 = jnp.concatenate([W1, ws1, wd1], axis=1)
    wd1p = jnp.concatenate([wd1, z8], axis=1)
    ws2, wd2 = fold(W2, as2, ad2)
    wcat2 = jnp.concatenate([W2, ws2, wd2], axis=1)
    wd2p = jnp.concatenate([wd2, z8], axis=1)
    ws3 = W3 @ as3[0]
    wd3 = W3 @ ad3[0]
    wcat3 = jnp.concatenate([W3, ws3[:, None],
                             jnp.zeros((128, 15), f32)], axis=1)
    wd3p = jnp.concatenate([wd3[:, None], jnp.zeros((128, 15), f32)], axis=1)

    dmat2 = (jnp.arange(16)[:, None] == jnp.arange(128)[None, :] // 16
             ).astype(f32)
    dmat3 = ((jnp.arange(16)[:, None] == 0) &
             (jnp.arange(64)[None, :] >= 0)).astype(f32)

    b1r = b1.reshape(1, 128)
    b2r = b2.reshape(1, 128)
    b3r = b3.reshape(1, 64)

    zinit = jnp.zeros((RD, 144), f32)

    z1, zd1 = _tc_proj(x, wcat1, wd1p)
    acc1 = _sc_edge_pass(z1, zd1, src3, dst3, zinit, 144, 128, 8, 1)
    z2, zd2 = _tc_comb(acc1, b1r, dmat2, wcat2, wd2p, 128, 128)
    acc2 = _sc_edge_pass(z2, zd2, src3, dst3, zinit, 144, 128, 8, 1)
    z3, zd3 = _tc_comb(acc2, b2r, dmat2, wcat3, wd3p, 128, 128)
    acc3 = _sc_edge_pass(z3, zd3, src3, dst3, zinit, 80, 64, 4, 4)
    return _tc_final(acc3, b3r, dmat3, 64, 64)


# speedup vs baseline: 1.1375x; 1.1375x over previous
import jax
import jax.numpy as jnp
from jax import lax
from jax.experimental import pallas as pl
from jax.experimental.pallas import tpu as pltpu
from jax.experimental.pallas import tpu_sc as plsc

NN = 10000
EE = 320000
NC, NS, LANES = 2, 16, 16
NW = NC * NS
ET = EE // NW
EB = 40
NB = ET // EB
RD = 624
TAIL = NN - NS * RD

f32 = jnp.float32
_PREC = lax.Precision.HIGHEST


def _sc_edge_pass(zs, zd, src3, dst3, zinit, rw, aoff, nv, vph):
    mesh = plsc.VectorSubcoreMesh(
        core_axis_name="c", subcore_axis_name="s",
        num_cores=NC, num_subcores=NS)

    def body(zs_hbm, zd_hbm, src_hbm, dst_hbm, zi_hbm, acc_hbm,
             acc_s, sidx, didx, rows0, rows1, rows2, zdr0, zdr1, zdr2,
             sg0, sg1, sg2, ss0, ss1, ss2):
        cid = lax.axis_index("c")
        sid = lax.axis_index("s")
        w = cid * NS + sid
        rows_ = (rows0, rows1, rows2)
        zdr_ = (zdr0, zdr1, zdr2)
        sg_ = (sg0, sg1, sg2)
        ss_ = (ss0, ss1, ss2)

        ia = pltpu.async_copy(src_hbm.at[w], sidx, sg0)
        ib = pltpu.async_copy(dst_hbm.at[w], didx, sg1)

        base_r = sid * RD
        pltpu.sync_copy(zi_hbm.at[pl.ds(0, RD), pl.ds(0, rw)],
                        acc_s.at[pl.ds(base_r, RD)])

        @pl.when(sid == 0)
        def _zero_tail():
            pltpu.sync_copy(zi_hbm.at[pl.ds(0, TAIL), pl.ds(0, rw)],
                            acc_s.at[pl.ds(NS * RD, TAIL)])
        ia.wait()
        ib.wait()
        plsc.subcore_barrier()

        heads = sorted(set(j // vph for j in range(nv)))
        hsplat = {h: jnp.full((LANES,), h, jnp.int32) for h in heads}

        def issue_gather(k, p):
            pltpu.async_copy(zs_hbm.at[sidx.at[k]], rows_[p], sg_[p])
            pltpu.async_copy(zd_hbm.at[didx.at[k]], zdr_[p], sg_[p])

        def wait_gather(p):
            pltpu.make_async_copy(zs_hbm.at[sidx.at[0]], rows_[p],
                                  sg_[p]).wait()
            pltpu.make_async_copy(zd_hbm.at[didx.at[0]], zdr_[p],
                                  sg_[p]).wait()

        def issue_scatter(k, p):
            pltpu.async_copy(rows_[p], acc_s.at[didx.at[k]], ss_[p],
                             add=True)

        def wait_scatter(p):
            pltpu.make_async_copy(rows_[p], acc_s.at[didx.at[0]],
                                  ss_[p]).wait()

        def compute(p):
            rp = rows_[p]
            zp = zdr_[p]

            def edge1(b):
                av = rp[b, pl.ds(aoff, LANES)] + zp[b, :]
                av = jnp.where(av > 0, av, av * 0.2)
                sv = jnp.exp(av)
                rp[b, pl.ds(aoff, LANES)] = sv
                ms = {h: sv.at[hsplat[h]].get(mode="promise_in_bounds")
                      for h in heads}
                for j in range(nv):
                    rp[b, pl.ds(j * LANES, LANES)] = (
                        ms[j // vph] * rp[b, pl.ds(j * LANES, LANES)])

            @plsc.parallel_loop(0, EB, unroll=4)
            def _edges(b):
                edge1(b)

        def step(k, b, issue_next, guard_first=False):
            wait_gather(b)
            compute(b)
            issue_scatter(k, b)
            if issue_next:
                bp = (b + 2) % 3
                if guard_first:
                    @pl.when(k >= 1)
                    def _w():
                        wait_scatter(bp)
                else:
                    wait_scatter(bp)
                issue_gather(k + 2, bp)

        issue_gather(0, 0)
        issue_gather(1, 1)

        def superstep(s, carry):
            step(3 * s, 0, True, guard_first=True)
            step(3 * s + 1, 1, True)
            step(3 * s + 2, 2, True)
            return carry
        nfull = (NB - 4) // 3
        lax.fori_loop(0, nfull, superstep, 0)
        step(NB - 4, 0, True)
        step(NB - 3, 1, True)
        step(NB - 2, 2, False)
        step(NB - 1, 0, False)
        wait_scatter(1)
        wait_scatter(2)
        wait_scatter(0)

        plsc.subcore_barrier()
        pltpu.sync_copy(acc_s.at[pl.ds(base_r, RD)],
                        acc_hbm.at[cid, pl.ds(base_r, RD)])

        @pl.when(sid == 0)
        def _drain_tail():
            pltpu.sync_copy(acc_s.at[pl.ds(NS * RD, TAIL)],
                            acc_hbm.at[cid, pl.ds(NS * RD, TAIL)])

    kfn = pl.kernel(
        body,
        out_type=jax.ShapeDtypeStruct((NC, NN, rw), f32),
        mesh=mesh,
        compiler_params=pltpu.CompilerParams(use_tc_tiling_on_sc=False),
        scratch_types=[
            pltpu.VMEM_SHARED((NN, rw), f32),
            pltpu.VMEM((NB, EB), jnp.int32),
            pltpu.VMEM((NB, EB), jnp.int32),
            pltpu.VMEM((EB, rw), f32),
            pltpu.VMEM((EB, rw), f32),
            pltpu.VMEM((EB, rw), f32),
            pltpu.VMEM((EB, 16), f32),
            pltpu.VMEM((EB, 16), f32),
            pltpu.VMEM((EB, 16), f32),
            pltpu.SemaphoreType.DMA,
            pltpu.SemaphoreType.DMA,
            pltpu.SemaphoreType.DMA,
            pltpu.SemaphoreType.DMA,
            pltpu.SemaphoreType.DMA,
            pltpu.SemaphoreType.DMA,
        ],
    )
    return kfn(zs, zd, src3, dst3, zinit)


def _tc_proj(x, wcat, wdp):
    k = wcat.shape[1]
    r = NN // 2

    def bdy(x_ref, wc_ref, wd_ref, z_ref, zd_ref):
        xb = x_ref[...]
        z_ref[...] = lax.dot_general(xb, wc_ref[...],
                                     (((1,), (0,)), ((), ())), precision=_PREC)
        zd_ref[...] = lax.dot_general(xb, wd_ref[...],
                                      (((1,), (0,)), ((), ())), precision=_PREC)

    return pl.pallas_call(
        bdy,
        grid=(NN // r,),
        in_specs=[pl.BlockSpec((r, x.shape[1]), lambda i: (i, 0)),
                  pl.BlockSpec(wcat.shape, lambda i: (0, 0)),
                  pl.BlockSpec(wdp.shape, lambda i: (0, 0))],
        out_specs=[pl.BlockSpec((r, k), lambda i: (i, 0)),
                   pl.BlockSpec((r, 16), lambda i: (i, 0))],
        out_shape=[jax.ShapeDtypeStruct((NN, k), f32),
                   jax.ShapeDtypeStruct((NN, 16), f32)],
    )(x, wcat, wdp)


def _tc_comb(acc, bias, dmat, wcat, wdp, hw, aoff):
    rwi = acc.shape[2]
    k = wcat.shape[1]
    r = NN // 2

    def bdy(a_ref, b_ref, dm_ref, wc_ref, wd_ref, z_ref, zd_ref):
        a = a_ref[0] + a_ref[1]
        h = a[:, :hw]
        d = a[:, aoff:aoff + 16]
        rec = 1.0 / (d + 1e-16)
        rb = lax.dot_general(rec, dm_ref[...],
                             (((1,), (0,)), ((), ())), precision=_PREC)
        xb = jnp.maximum(h * rb + b_ref[...], 0.0)
        z_ref[...] = lax.dot_general(xb, wc_ref[...],
                                     (((1,), (0,)), ((), ())), precision=_PREC)
        zd_ref[...] = lax.dot_general(xb, wd_ref[...],
                                      (((1,), (0,)), ((), ())), precision=_PREC)

    return pl.pallas_call(
        bdy,
        grid=(NN // r,),
        in_specs=[pl.BlockSpec((NC, r, rwi), lambda i: (0, i, 0)),
                  pl.BlockSpec((1, hw), lambda i: (0, 0)),
                  pl.BlockSpec((16, hw), lambda i: (0, 0)),
                  pl.BlockSpec(wcat.shape, lambda i: (0, 0)),
                  pl.BlockSpec(wdp.shape, lambda i: (0, 0))],
        out_specs=[pl.BlockSpec((r, k), lambda i: (i, 0)),
                   pl.BlockSpec((r, 16), lambda i: (i, 0))],
        out_shape=[jax.ShapeDtypeStruct((NN, k), f32),
                   jax.ShapeDtypeStruct((NN, 16), f32)],
    )(acc, bias, dmat, wcat, wdp)


def _tc_final(acc, bias, dmat, hw, aoff):
    rwi = acc.shape[2]
    r = NN // 2

    def bdy(a_ref, b_ref, dm_ref, o_ref):
        a = a_ref[0] + a_ref[1]
        h = a[:, :hw]
        d = a[:, aoff:aoff + 16]
        rec = 1.0 / (d + 1e-16)
        rb = lax.dot_general(rec, dm_ref[...],
                             (((1,), (0,)), ((), ())), precision=_PREC)
        o = h * rb + b_ref[...]
        m = jnp.max(o, axis=1, keepdims=True)
        e = o - m
        s = jnp.sum(jnp.exp(e), axis=1, keepdims=True)
        o_ref[...] = e - jnp.log(s)

    return pl.pallas_call(
        bdy,
        grid=(NN // r,),
        in_specs=[pl.BlockSpec((NC, r, rwi), lambda i: (0, i, 0)),
                  pl.BlockSpec((1, hw), lambda i: (0, 0)),
                  pl.BlockSpec((16, hw), lambda i: (0, 0))],
        out_specs=pl.BlockSpec((r, hw), lambda i: (i, 0)),
        out_shape=jax.ShapeDtypeStruct((NN, hw), f32),
    )(acc, bias, dmat)


def kernel(x, edge_index, W1, as1, ad1, b1, W2, as2, ad2, b2, W3, as3, ad3, b3):
    src3 = edge_index[0].reshape(NW, NB, EB)
    dst3 = edge_index[1].reshape(NW, NB, EB)

    gh = (jnp.arange(128)[:, None] // 16 == jnp.arange(8)[None, :]).astype(f32)

    def fold(W, a_s, a_d):
        asf = a_s.reshape(-1)
        adf = a_d.reshape(-1)
        ws = (W * asf[None, :]) @ gh
        wd = (W * adf[None, :]) @ gh
        return ws, wd

    z8 = jnp.zeros((128, 8), f32)
    ws1, wd1 = fold(W1, as1, ad1)
    wcat1 = jnp.concatenate([W1, ws1, wd1], axis=1)
    wd1p = jnp.concatenate([wd1, z8], axis=1)
    ws2, wd2 = fold(W2, as2, ad2)
    wcat2 = jnp.concatenate([W2, ws2, wd2], axis=1)
    wd2p = jnp.concatenate([wd2, z8], axis=1)
    ws3 = W3 @ as3[0]
    wd3 = W3 @ ad3[0]
    wcat3 = jnp.concatenate([W3, ws3[:, None],
                             jnp.zeros((128, 15), f32)], axis=1)
    wd3p = jnp.concatenate([wd3[:, None], jnp.zeros((128, 15), f32)], axis=1)

    dmat2 = (jnp.arange(16)[:, None] == jnp.arange(128)[None, :] // 16
             ).astype(f32)
    dmat3 = ((jnp.arange(16)[:, None] == 0) &
             (jnp.arange(64)[None, :] >= 0)).astype(f32)

    b1r = b1.reshape(1, 128)
    b2r = b2.reshape(1, 128)
    b3r = b3.reshape(1, 64)

    zinit = jnp.zeros((RD, 144), f32)

    z1, zd1 = _tc_proj(x, wcat1, wd1p)
    acc1 = _sc_edge_pass(z1, zd1, src3, dst3, zinit, 144, 128, 8, 1)
    z2, zd2 = _tc_comb(acc1, b1r, dmat2, wcat2, wd2p, 128, 128)
    acc2 = _sc_edge_pass(z2, zd2, src3, dst3, zinit, 144, 128, 8, 1)
    z3, zd3 = _tc_comb(acc2, b2r, dmat2, wcat3, wd3p, 128, 128)
    acc3 = _sc_edge_pass(z3, zd3, src3, dst3, zinit, 80, 64, 4, 4)
    return _tc_final(acc3, b3r, dmat3, 64, 64)
